# baseline (speedup 1.0000x reference)
import functools

import jax
import jax.numpy as jnp
from jax import lax
from jax.experimental import pallas as pl
from jax.experimental.pallas import tpu as pltpu
from jax.experimental.pallas import tpu_sc as plsc

D = 64
B = 4096 * 200
NC, NS = 2, 16
NW = NC * NS
B_PER_W = B // NW
CHUNK = 512
N_CHUNKS = B_PER_W // CHUNK

_mesh = plsc.VectorSubcoreMesh(core_axis_name="c", subcore_axis_name="s")


@functools.partial(
    pl.kernel,
    mesh=_mesh,
    out_type=jax.ShapeDtypeStruct((B, D), jnp.float32),
    scratch_types=[
        pltpu.VMEM((CHUNK,), jnp.int32),
        pltpu.VMEM((CHUNK, D), jnp.float32),
        pltpu.SemaphoreType.DMA,
    ],
    compiler_params=pltpu.CompilerParams(use_tc_tiling_on_sc=False),
)
def _emb_lookup(idx_hbm, table_hbm, out_hbm, idx_v, rows_v, sem):
    wid = lax.axis_index("s") * NC + lax.axis_index("c")
    base = wid * B_PER_W

    def body(i, carry):
        off = base + i * CHUNK
        pltpu.sync_copy(idx_hbm.at[pl.ds(off, CHUNK)], idx_v)
        pltpu.async_copy(table_hbm.at[idx_v], rows_v, sem).wait()
        pltpu.sync_copy(rows_v, out_hbm.at[pl.ds(off, CHUNK)])
        return carry

    lax.fori_loop(0, N_CHUNKS, body, 0)


def kernel(token_ids, embedding):
    flat = token_ids.reshape(-1).astype(jnp.int32)
    out = _emb_lookup(flat, embedding)
    return out.reshape(token_ids.shape + (D,))


# speedup vs baseline: 1.0430x; 1.0430x over previous
import functools

import jax
import jax.numpy as jnp
from jax import lax
from jax.experimental import pallas as pl
from jax.experimental.pallas import tpu as pltpu
from jax.experimental.pallas import tpu_sc as plsc

D = 64
B = 4096 * 200
NC, NS = 2, 16
NW = NC * NS
B_PER_W = B // NW
CHUNK = 256
NBUF = 4
N_CHUNKS = B_PER_W // CHUNK
N_GROUPS = N_CHUNKS // NBUF

_mesh = plsc.VectorSubcoreMesh(core_axis_name="c", subcore_axis_name="s")


@functools.partial(
    pl.kernel,
    mesh=_mesh,
    out_type=jax.ShapeDtypeStruct((B, D), jnp.float32),
    scratch_types=[
        pltpu.VMEM((B_PER_W,), jnp.int32),
        pltpu.VMEM((NBUF, CHUNK, D), jnp.float32),
        pltpu.SemaphoreType.DMA,
        [pltpu.SemaphoreType.DMA] * NBUF,
        [pltpu.SemaphoreType.DMA] * NBUF,
    ],
    compiler_params=pltpu.CompilerParams(use_tc_tiling_on_sc=False),
)
def _emb_lookup(idx_hbm, table_hbm, out_hbm, idx_v, rows_v, isem, gsems, ssems):
    wid = lax.axis_index("s") * NC + lax.axis_index("c")
    base = wid * B_PER_W

    pltpu.async_copy(idx_hbm.at[pl.ds(base, B_PER_W)], idx_v, isem).wait()

    def idx_slice(chunk):
        return idx_v.at[pl.ds(chunk * CHUNK, CHUNK)]

    def start_gather(chunk, b):
        pltpu.async_copy(table_hbm.at[idx_slice(chunk)], rows_v.at[b], gsems[b])

    def wait_gather(chunk, b):
        pltpu.make_async_copy(
            table_hbm.at[idx_slice(chunk)], rows_v.at[b], gsems[b]
        ).wait()

    def out_slice(chunk):
        return out_hbm.at[pl.ds(base + chunk * CHUNK, CHUNK)]

    def start_store(chunk, b):
        pltpu.async_copy(rows_v.at[b], out_slice(chunk), ssems[b])

    def wait_store(chunk, b):
        pltpu.make_async_copy(rows_v.at[b], out_slice(chunk), ssems[b]).wait()

    for b in range(NBUF):
        start_gather(b, b)

    def group(g, carry):
        c0 = g * NBUF
        for b in range(NBUF):
            wait_gather(c0 + b, b)
            start_store(c0 + b, b)
        for b in range(NBUF):
            wait_store(c0 + b, b)
            start_gather(c0 + NBUF + b, b)
        return carry

    lax.fori_loop(0, N_GROUPS - 1, group, 0)

    c0 = (N_GROUPS - 1) * NBUF
    for b in range(NBUF):
        wait_gather(c0 + b, b)
        start_store(c0 + b, b)
    for b in range(NBUF):
        wait_store(c0 + b, b)


def kernel(token_ids, embedding):
    flat = token_ids.reshape(-1).astype(jnp.int32)
    out = _emb_lookup(flat, embedding)
    return out.reshape(token_ids.shape + (D,))
